# baseline (speedup 1.0000x reference)
import jax
import jax.numpy as jnp
import numpy as np
from jax.experimental import pallas as pl
from jax.experimental.pallas import tpu as pltpu

N_HEADS = 12
EPS = 1e-05
CHUNK_SIZE = 16
DIM = 768
HIDDEN = 2048
HD = DIM // N_HEADS

BLK = 256
BLK_Q = 256

INTERPRET = False


def _rms(x, w):
    return x * jax.lax.rsqrt(jnp.mean(x * x, axis=-1, keepdims=True) + EPS) * w


def _rotate_half(x):
    x1, x2 = jnp.split(x, 2, axis=-1)
    return jnp.concatenate([-x2, x1], axis=-1)


def _qkv_kernel(x_ref, anw_ref, w_ref, cosf_ref, sinf_ref, q_ref, k_ref, v_ref):
    h = _rms(x_ref[...], anw_ref[...]).astype(jnp.bfloat16)
    p = jnp.dot(h, w_ref[...], preferred_element_type=jnp.float32)
    cosf = cosf_ref[...]
    sinf = sinf_ref[...]
    q = p[:, 0 * DIM:1 * DIM] * cosf + p[:, 1 * DIM:2 * DIM] * sinf
    k = p[:, 2 * DIM:3 * DIM] * cosf + p[:, 3 * DIM:4 * DIM] * sinf
    q_ref[...] = q.astype(jnp.bfloat16)
    k_ref[...] = k.astype(jnp.bfloat16)
    v_ref[...] = p[:, 4 * DIM:5 * DIM].astype(jnp.bfloat16)


BLK_K = 256


def _attn_kernel(q_ref, k_ref, v_ref, o_ref):
    qi = pl.program_id(1)
    scale = 1.0 / np.sqrt(HD)
    q_pos = qi * BLK_Q + jax.lax.broadcasted_iota(jnp.int32, (BLK_Q, BLK_K), 0)
    k_iota = jax.lax.broadcasted_iota(jnp.int32, (BLK_Q, BLK_K), 1)
    for hh in range(N_HEADS):
        sl = slice(hh * HD, (hh + 1) * HD)
        qh = q_ref[0, :, sl]

        def body(j, carry):
            m, l, acc = carry
            kh = k_ref[0, pl.ds(j * BLK_K, BLK_K), sl]
            vh = v_ref[0, pl.ds(j * BLK_K, BLK_K), sl]
            s = jax.lax.dot_general(qh, kh, (((1,), (1,)), ((), ())),
                                    preferred_element_type=jnp.float32) * scale
            s = jnp.where(j * BLK_K + k_iota <= q_pos, s, -1e9)
            m_new = jnp.maximum(m, jnp.max(s, axis=-1, keepdims=True))
            alpha = jnp.exp(m - m_new)
            e = jnp.exp(s - m_new)
            l = l * alpha + jnp.sum(e, axis=-1, keepdims=True)
            acc = acc * alpha + jnp.dot(e.astype(jnp.bfloat16), vh,
                                        preferred_element_type=jnp.float32)
            return m_new, l, acc

        m0 = jnp.full((BLK_Q, 1), -jnp.inf, jnp.float32)
        l0 = jnp.zeros((BLK_Q, 1), jnp.float32)
        a0 = jnp.zeros((BLK_Q, HD), jnp.float32)
        m, l, acc = jax.lax.fori_loop(0, qi + 1, body, (m0, l0, a0))
        o_ref[0, :, sl] = (acc / l).astype(jnp.bfloat16)


def _ffn_kernel(x_ref, o_ref, wo_ref, fnw_ref, w13_ref, w2_ref, out_ref):
    x2 = x_ref[...] + jnp.dot(o_ref[...], wo_ref[...],
                              preferred_element_type=jnp.float32)
    h2 = _rms(x2, fnw_ref[...]).astype(jnp.bfloat16)
    a = jnp.dot(h2, w13_ref[...], preferred_element_type=jnp.float32)
    ff = (jax.nn.silu(a[:, :HIDDEN]) * a[:, HIDDEN:]).astype(jnp.bfloat16)
    out_ref[...] = x2 + jnp.dot(ff, w2_ref[...],
                                preferred_element_type=jnp.float32)


def _final_kernel(x_ref, nw_ref, xn_ref, comp_ref):
    xn = _rms(x_ref[...], nw_ref[...])
    xn_ref[...] = xn
    for j in range(BLK // CHUNK_SIZE):
        comp_ref[j, :] = xn[j * CHUNK_SIZE, :]


def kernel(x, cos, sin, layers_attn_norm, layers_wq, layers_wk, layers_wv,
           layers_wo, layers_ffn_norm, layers_w1, layers_w2, layers_w3,
           norm_w):
    B, L, D = x.shape
    n_layers = layers_wq.shape[0]
    R = B * L
    nblk = R // BLK

    cosf = jnp.tile(cos, (1, N_HEADS))
    sinf = jnp.tile(sin, (1, N_HEADS))

    row_spec = pl.BlockSpec((BLK, DIM), lambda i: (i, 0))
    cs_spec = pl.BlockSpec((BLK, DIM), lambda i: (i % (L // BLK), 0))
    vec_spec = pl.BlockSpec((1, DIM), lambda i: (0, 0))

    xf = x.reshape(R, D)
    for li in range(n_layers):
        wq, wk = layers_wq[li], layers_wk[li]
        wqr = _rotate_half(wq.reshape(D, N_HEADS, HD)).reshape(D, D)
        wkr = _rotate_half(wk.reshape(D, N_HEADS, HD)).reshape(D, D)
        wcat = jnp.concatenate([wq, wqr, wk, wkr, layers_wv[li]],
                               axis=1).astype(jnp.bfloat16)

        q, k, v = pl.pallas_call(
            _qkv_kernel,
            grid=(nblk,),
            in_specs=[
                row_spec,
                vec_spec,
                pl.BlockSpec((D, 5 * DIM), lambda i: (0, 0)),
                cs_spec,
                cs_spec,
            ],
            out_specs=[row_spec, row_spec, row_spec],
            out_shape=[jax.ShapeDtypeStruct((R, D), jnp.bfloat16)] * 3,
            compiler_params=pltpu.CompilerParams(
                dimension_semantics=("parallel",)),
            interpret=INTERPRET,
        )(xf, layers_attn_norm[li][None], wcat, cosf, sinf)

        o = pl.pallas_call(
            _attn_kernel,
            grid=(B, L // BLK_Q),
            in_specs=[
                pl.BlockSpec((1, BLK_Q, D), lambda b, i: (b, i, 0)),
                pl.BlockSpec((1, L, D), lambda b, i: (b, 0, 0)),
                pl.BlockSpec((1, L, D), lambda b, i: (b, 0, 0)),
            ],
            out_specs=pl.BlockSpec((1, BLK_Q, D), lambda b, i: (b, i, 0)),
            out_shape=jax.ShapeDtypeStruct((B, L, D), jnp.bfloat16),
            compiler_params=pltpu.CompilerParams(
                dimension_semantics=("parallel", "parallel")),
            interpret=INTERPRET,
        )(q.reshape(B, L, D), k.reshape(B, L, D), v.reshape(B, L, D))

        w13 = jnp.concatenate([layers_w1[li], layers_w3[li]],
                              axis=1).astype(jnp.bfloat16)
        xf = pl.pallas_call(
            _ffn_kernel,
            grid=(nblk,),
            in_specs=[
                row_spec,
                row_spec,
                pl.BlockSpec((D, D), lambda i: (0, 0)),
                vec_spec,
                pl.BlockSpec((D, 2 * HIDDEN), lambda i: (0, 0)),
                pl.BlockSpec((HIDDEN, D), lambda i: (0, 0)),
            ],
            out_specs=row_spec,
            out_shape=jax.ShapeDtypeStruct((R, D), jnp.float32),
            compiler_params=pltpu.CompilerParams(
                dimension_semantics=("parallel",)),
            interpret=INTERPRET,
        )(xf, o.reshape(R, D), layers_wo[li].astype(jnp.bfloat16),
          layers_ffn_norm[li][None], w13, layers_w2[li].astype(jnp.bfloat16))

    S = L // CHUNK_SIZE
    xn_f, comp_f = pl.pallas_call(
        _final_kernel,
        grid=(nblk,),
        in_specs=[row_spec, vec_spec],
        out_specs=[
            row_spec,
            pl.BlockSpec((BLK // CHUNK_SIZE, DIM), lambda i: (i, 0)),
        ],
        out_shape=[
            jax.ShapeDtypeStruct((R, D), jnp.float32),
            jax.ShapeDtypeStruct((R // CHUNK_SIZE, D), jnp.float32),
        ],
        compiler_params=pltpu.CompilerParams(
            dimension_semantics=("parallel",)),
        interpret=INTERPRET,
    )(xf, norm_w[None])

    xn = xn_f.reshape(B, L, D)
    compressed_x = comp_f.reshape(B, S, D)
    starts = jnp.arange(0, L, CHUNK_SIZE)
    boundary_positions = jnp.broadcast_to(starts[None, :], (B, S))
    counts = jnp.full((B,), S, dtype=jnp.int32)
    avg_chunk_size = float(L) / float(S)
    return (xn, compressed_x, boundary_positions, counts, avg_chunk_size)


# speedup vs baseline: 1.9057x; 1.9057x over previous
import jax
import jax.numpy as jnp
import numpy as np
from jax.experimental import pallas as pl
from jax.experimental.pallas import tpu as pltpu

N_HEADS = 12
EPS = 1e-05
CHUNK_SIZE = 16
DIM = 768
HIDDEN = 2048
HD = DIM // N_HEADS

BLK = 256
BLK_Q = 256

INTERPRET = False


def _rms(x, w):
    return x * jax.lax.rsqrt(jnp.mean(x * x, axis=-1, keepdims=True) + EPS) * w


def _rotate_half(x):
    x1, x2 = jnp.split(x, 2, axis=-1)
    return jnp.concatenate([-x2, x1], axis=-1)


def _qkv_kernel(x_ref, anw_ref, w_ref, cosf_ref, sinf_ref, q_ref, k_ref, v_ref):
    h = _rms(x_ref[...], anw_ref[...]).astype(jnp.bfloat16)
    p = jnp.dot(h, w_ref[...], preferred_element_type=jnp.float32)
    cosf = cosf_ref[...]
    sinf = sinf_ref[...]
    q = p[:, 0 * DIM:1 * DIM] * cosf + p[:, 1 * DIM:2 * DIM] * sinf
    k = p[:, 2 * DIM:3 * DIM] * cosf + p[:, 3 * DIM:4 * DIM] * sinf
    q_ref[...] = q.astype(jnp.bfloat16)
    k_ref[...] = k.astype(jnp.bfloat16)
    v_ref[...] = p[:, 4 * DIM:5 * DIM].astype(jnp.bfloat16)


def _attn_kernel(q_ref, k_ref, v_ref, o_ref):
    L = k_ref.shape[1]
    qi = pl.program_id(1)
    q_pos = qi * BLK_Q + jax.lax.broadcasted_iota(jnp.int32, (BLK_Q, L), 0)
    k_pos = jax.lax.broadcasted_iota(jnp.int32, (BLK_Q, L), 1)
    neg = jnp.where(k_pos <= q_pos, 0.0, -1e9)
    scale = 1.0 / np.sqrt(HD)
    for hh in range(N_HEADS):
        sl = slice(hh * HD, (hh + 1) * HD)
        qh = q_ref[0, :, sl]
        kh = k_ref[0, :, sl]
        s = jax.lax.dot_general(qh, kh, (((1,), (1,)), ((), ())),
                                preferred_element_type=jnp.float32)
        s = s * scale + neg
        m = jnp.max(s, axis=-1, keepdims=True)
        e = jnp.exp(s - m)
        p = e.astype(jnp.bfloat16)
        r = 1.0 / jnp.sum(e, axis=-1, keepdims=True)
        o_ref[0, :, sl] = (jnp.dot(p, v_ref[0, :, sl],
                                   preferred_element_type=jnp.float32)
                           * r).astype(jnp.bfloat16)


def _ffn_kernel(x_ref, o_ref, wo_ref, fnw_ref, w13_ref, w2_ref, out_ref):
    x2 = x_ref[...] + jnp.dot(o_ref[...], wo_ref[...],
                              preferred_element_type=jnp.float32)
    h2 = _rms(x2, fnw_ref[...]).astype(jnp.bfloat16)
    a = jnp.dot(h2, w13_ref[...], preferred_element_type=jnp.float32)
    ff = (jax.nn.silu(a[:, :HIDDEN]) * a[:, HIDDEN:]).astype(jnp.bfloat16)
    out_ref[...] = x2 + jnp.dot(ff, w2_ref[...],
                                preferred_element_type=jnp.float32)


def _final_kernel(x_ref, nw_ref, xn_ref, comp_ref):
    xn = _rms(x_ref[...], nw_ref[...])
    xn_ref[...] = xn
    for j in range(BLK // CHUNK_SIZE):
        comp_ref[j, :] = xn[j * CHUNK_SIZE, :]


def kernel(x, cos, sin, layers_attn_norm, layers_wq, layers_wk, layers_wv,
           layers_wo, layers_ffn_norm, layers_w1, layers_w2, layers_w3,
           norm_w):
    B, L, D = x.shape
    n_layers = layers_wq.shape[0]
    R = B * L
    nblk = R // BLK

    cosf = jnp.tile(cos, (1, N_HEADS))
    sinf = jnp.tile(sin, (1, N_HEADS))

    row_spec = pl.BlockSpec((BLK, DIM), lambda i: (i, 0))
    cs_spec = pl.BlockSpec((BLK, DIM), lambda i: (i % (L // BLK), 0))
    vec_spec = pl.BlockSpec((1, DIM), lambda i: (0, 0))

    xf = x.reshape(R, D)
    for li in range(n_layers):
        wq, wk = layers_wq[li], layers_wk[li]
        wqr = _rotate_half(wq.reshape(D, N_HEADS, HD)).reshape(D, D)
        wkr = _rotate_half(wk.reshape(D, N_HEADS, HD)).reshape(D, D)
        wcat = jnp.concatenate([wq, wqr, wk, wkr, layers_wv[li]],
                               axis=1).astype(jnp.bfloat16)

        q, k, v = pl.pallas_call(
            _qkv_kernel,
            grid=(nblk,),
            in_specs=[
                row_spec,
                vec_spec,
                pl.BlockSpec((D, 5 * DIM), lambda i: (0, 0)),
                cs_spec,
                cs_spec,
            ],
            out_specs=[row_spec, row_spec, row_spec],
            out_shape=[jax.ShapeDtypeStruct((R, D), jnp.bfloat16)] * 3,
            compiler_params=pltpu.CompilerParams(
                dimension_semantics=("parallel",)),
            interpret=INTERPRET,
        )(xf, layers_attn_norm[li][None], wcat, cosf, sinf)

        o = pl.pallas_call(
            _attn_kernel,
            grid=(B, L // BLK_Q),
            in_specs=[
                pl.BlockSpec((1, BLK_Q, D), lambda b, i: (b, i, 0)),
                pl.BlockSpec((1, L, D), lambda b, i: (b, 0, 0)),
                pl.BlockSpec((1, L, D), lambda b, i: (b, 0, 0)),
            ],
            out_specs=pl.BlockSpec((1, BLK_Q, D), lambda b, i: (b, i, 0)),
            out_shape=jax.ShapeDtypeStruct((B, L, D), jnp.bfloat16),
            compiler_params=pltpu.CompilerParams(
                dimension_semantics=("parallel", "parallel")),
            interpret=INTERPRET,
        )(q.reshape(B, L, D), k.reshape(B, L, D), v.reshape(B, L, D))

        w13 = jnp.concatenate([layers_w1[li], layers_w3[li]],
                              axis=1).astype(jnp.bfloat16)
        xf = pl.pallas_call(
            _ffn_kernel,
            grid=(nblk,),
            in_specs=[
                row_spec,
                row_spec,
                pl.BlockSpec((D, D), lambda i: (0, 0)),
                vec_spec,
                pl.BlockSpec((D, 2 * HIDDEN), lambda i: (0, 0)),
                pl.BlockSpec((HIDDEN, D), lambda i: (0, 0)),
            ],
            out_specs=row_spec,
            out_shape=jax.ShapeDtypeStruct((R, D), jnp.float32),
            compiler_params=pltpu.CompilerParams(
                dimension_semantics=("parallel",)),
            interpret=INTERPRET,
        )(xf, o.reshape(R, D), layers_wo[li].astype(jnp.bfloat16),
          layers_ffn_norm[li][None], w13, layers_w2[li].astype(jnp.bfloat16))

    S = L // CHUNK_SIZE
    xn_f, comp_f = pl.pallas_call(
        _final_kernel,
        grid=(nblk,),
        in_specs=[row_spec, vec_spec],
        out_specs=[
            row_spec,
            pl.BlockSpec((BLK // CHUNK_SIZE, DIM), lambda i: (i, 0)),
        ],
        out_shape=[
            jax.ShapeDtypeStruct((R, D), jnp.float32),
            jax.ShapeDtypeStruct((R // CHUNK_SIZE, D), jnp.float32),
        ],
        compiler_params=pltpu.CompilerParams(
            dimension_semantics=("parallel",)),
        interpret=INTERPRET,
    )(xf, norm_w[None])

    xn = xn_f.reshape(B, L, D)
    compressed_x = comp_f.reshape(B, S, D)
    starts = jnp.arange(0, L, CHUNK_SIZE)
    boundary_positions = jnp.broadcast_to(starts[None, :], (B, S))
    counts = jnp.full((B,), S, dtype=jnp.int32)
    avg_chunk_size = float(L) / float(S)
    return (xn, compressed_x, boundary_positions, counts, avg_chunk_size)
